# baseline (speedup 1.0000x reference)
import functools

import jax
import jax.numpy as jnp
from jax import lax
from jax.experimental import pallas as pl
from jax.experimental.pallas import tpu as pltpu
from jax.experimental.pallas import tpu_sc as plsc

N = 10000
D = 128
G = 64
NPAD = 10112
RPT = NPAD // 16
CH = 128
TOT = 2560
EPAD = TOT * CH
K0 = 48
K1 = 112
KMAX = 112
IDXR = 16
NSTG = KMAX // IDXR
BLK = 1000

_mesh = lambda: plsc.VectorSubcoreMesh(core_axis_name="c", subcore_axis_name="s")


@functools.partial(
    pl.kernel,
    mesh=_mesh(),
    out_type=jax.ShapeDtypeStruct((2 * NPAD,), jnp.float32),
    scratch_types=[
        pltpu.VMEM_SHARED((NPAD,), jnp.float32),
        pltpu.VMEM((KMAX, CH), jnp.int32),
        pltpu.VMEM((CH,), jnp.float32),
        pltpu.VMEM((640,), jnp.float32),
    ],
)
def _sc_deg(dst_hbm, out_hbm, acc, dst_v, ones_v, zb_v):
    c = lax.axis_index("c")
    s = lax.axis_index("s")
    wid = c * 16 + s
    kc = jnp.where(c == 0, K0, K1)

    def fill(i, _):
        ones_v[pl.ds(i * 16, 16)] = jnp.ones((16,), jnp.float32)
        zb_v[pl.ds(i * 16, 16)] = jnp.zeros((16,), jnp.float32)
        return 0

    lax.fori_loop(0, 8, fill, 0)

    def fillz(i, _):
        zb_v[pl.ds(i * 16, 16)] = jnp.zeros((16,), jnp.float32)
        return 0

    lax.fori_loop(8, 40, fillz, 0)

    pltpu.sync_copy(dst_hbm.at[wid], dst_v)
    base = s * RPT
    pltpu.sync_copy(zb_v.at[pl.ds(0, RPT)], acc.at[pl.ds(base, RPT)])
    plsc.subcore_barrier()

    def body(j, _):
        pltpu.sync_copy(ones_v, acc.at[dst_v.at[j]], add=True)
        return 0

    lax.fori_loop(0, kc, body, 0)
    plsc.subcore_barrier()

    pltpu.sync_copy(acc.at[pl.ds(base, RPT)], zb_v.at[pl.ds(0, RPT)])
    pltpu.sync_copy(zb_v.at[pl.ds(0, RPT)], out_hbm.at[pl.ds(c * NPAD + base, RPT)])


@functools.partial(
    pl.kernel,
    mesh=_mesh(),
    out_type=jax.ShapeDtypeStruct((2 * NPAD, D), jnp.float32),
    scratch_types=[
        pltpu.VMEM_SHARED((NPAD, D), jnp.float32),
        pltpu.VMEM((IDXR, CH), jnp.int32),
        pltpu.VMEM((IDXR, CH), jnp.int32),
        pltpu.VMEM((CH, D), jnp.float32),
        pltpu.VMEM((CH, D), jnp.float32),
        pltpu.SemaphoreType.DMA,
        pltpu.SemaphoreType.DMA,
    ],
)
def _sc_layer(src_hbm, dst_hbm, table_hbm, out_hbm, acc, src_v, dst_v,
              rows0_v, rows1_v, sem0, sem1):
    c = lax.axis_index("c")
    s = lax.axis_index("s")
    wid = c * 16 + s
    base = s * RPT

    def zrow(i, _):
        r = i // 8
        g = i % 8
        rows0_v[r, pl.ds(g * 16, 16)] = jnp.zeros((16,), jnp.float32)
        return 0

    lax.fori_loop(0, CH * 8, zrow, 0)
    for k in range(4):
        pltpu.sync_copy(rows0_v, acc.at[pl.ds(base + k * CH, CH)])
    tail = RPT - 4 * CH
    pltpu.sync_copy(rows0_v.at[pl.ds(0, tail)],
                    acc.at[pl.ds(base + 4 * CH, tail)])
    plsc.subcore_barrier()

    kc = jnp.where(c == 0, K0, K1)
    for st in range(NSTG):
        @pl.when(st * IDXR < kc)
        def _stage():
            pltpu.sync_copy(src_hbm.at[wid, pl.ds(st * IDXR, IDXR)], src_v)
            pltpu.sync_copy(dst_hbm.at[wid, pl.ds(st * IDXR, IDXR)], dst_v)
            pltpu.async_copy(table_hbm.at[src_v.at[0]], rows0_v, sem0)
            pltpu.async_copy(table_hbm.at[src_v.at[1]], rows1_v, sem1)

            def body(i, _):
                j0 = 2 * i
                j1 = j0 + 1
                pltpu.make_async_copy(
                    table_hbm.at[src_v.at[j0]], rows0_v, sem0).wait()
                pltpu.sync_copy(rows0_v, acc.at[dst_v.at[j0]], add=True)

                @pl.when(j0 + 2 < IDXR)
                def _():
                    pltpu.async_copy(
                        table_hbm.at[src_v.at[j0 + 2]], rows0_v, sem0)

                pltpu.make_async_copy(
                    table_hbm.at[src_v.at[j1]], rows1_v, sem1).wait()
                pltpu.sync_copy(rows1_v, acc.at[dst_v.at[j1]], add=True)

                @pl.when(j1 + 2 < IDXR)
                def _():
                    pltpu.async_copy(
                        table_hbm.at[src_v.at[j1 + 2]], rows1_v, sem1)

                return 0

            lax.fori_loop(0, IDXR // 2, body, 0)
    plsc.subcore_barrier()

    for k in range(4):
        pltpu.sync_copy(acc.at[pl.ds(base + k * CH, CH)], rows0_v)
        pltpu.sync_copy(rows0_v, out_hbm.at[pl.ds(c * NPAD + base + k * CH, CH)])
    pltpu.sync_copy(acc.at[pl.ds(base + 4 * CH, tail)],
                    rows0_v.at[pl.ds(0, tail)])
    pltpu.sync_copy(rows0_v.at[pl.ds(0, tail)],
                    out_hbm.at[pl.ds(c * NPAD + base + 4 * CH, tail)])


def _dense0_body(x_ref, w_ref, d0_ref, d1_ref, out_ref, dinv_ref):
    dinv = lax.rsqrt(1.0 + d0_ref[...] + d1_ref[...])
    out_ref[...] = dinv * jnp.dot(
        x_ref[...], w_ref[...], preferred_element_type=jnp.float32)
    dinv_ref[...] = dinv


def _dense_mid_body(p0_ref, p1_ref, xws_ref, dinv_ref, b_ref, w_ref, out_ref):
    dinv = dinv_ref[...]
    h = jnp.maximum(
        dinv * (p0_ref[...] + p1_ref[...] + xws_ref[...]) + b_ref[...], 0.0)
    out_ref[...] = dinv * jnp.dot(
        h, w_ref[...], preferred_element_type=jnp.float32)


def _dense_last_body(p0_ref, p1_ref, xws_ref, dinv_ref, b_ref, batch_ref, out_ref):
    dinv = dinv_ref[...]
    h = jnp.maximum(
        dinv * (p0_ref[...] + p1_ref[...] + xws_ref[...]) + b_ref[...], 0.0)
    gids = lax.broadcasted_iota(jnp.int32, (BLK, G), 1)
    onehot = (batch_ref[...] == gids).astype(jnp.float32)
    contrib = lax.dot_general(
        onehot, h, (((0,), (0,)), ((), ())),
        preferred_element_type=jnp.float32)

    @pl.when(pl.program_id(0) == 0)
    def _():
        out_ref[...] = jnp.zeros_like(out_ref)

    out_ref[...] += contrib


_row_spec = pl.BlockSpec((BLK, D), lambda i: (i, 0))
_col_spec = pl.BlockSpec((BLK, 1), lambda i: (i, 0))
_w_spec = pl.BlockSpec((D, D), lambda i: (0, 0))
_b_spec = pl.BlockSpec((1, D), lambda i: (0, 0))
_p_spec = pl.BlockSpec((2, BLK, D), lambda i: (0, i, 0))

_GRID = N // BLK

_dense0 = pl.pallas_call(
    _dense0_body,
    grid=(_GRID,),
    in_specs=[_row_spec, _w_spec, _col_spec, _col_spec],
    out_specs=[_row_spec, _col_spec],
    out_shape=[
        jax.ShapeDtypeStruct((N, D), jnp.float32),
        jax.ShapeDtypeStruct((N, 1), jnp.float32),
    ],
)

_dense_mid = pl.pallas_call(
    _dense_mid_body,
    grid=(_GRID,),
    in_specs=[_row_spec, _row_spec, _row_spec, _col_spec, _b_spec, _w_spec],
    out_specs=_row_spec,
    out_shape=jax.ShapeDtypeStruct((N, D), jnp.float32),
)

_dense_last = pl.pallas_call(
    _dense_last_body,
    grid=(_GRID,),
    in_specs=[_row_spec, _row_spec, _row_spec, _col_spec, _b_spec,
              pl.BlockSpec((BLK, 1), lambda i: (i, 0))],
    out_specs=pl.BlockSpec((G, D), lambda i: (0, 0)),
    out_shape=jax.ShapeDtypeStruct((G, D), jnp.float32),
)


@jax.jit
def kernel(x, edge_index, batch, W1, b1, W2, b2, W3, b3):
    src = edge_index[0]
    dst = edge_index[1]
    pad = EPAD - src.shape[0]
    src_f = jnp.concatenate(
        [src, jnp.zeros((pad,), jnp.int32)]).reshape(TOT, CH)
    dummy = N + (jnp.arange(pad, dtype=jnp.int32) % (NPAD - N))
    dst_f = jnp.concatenate([dst, dummy]).reshape(TOT, CH)

    def split(a):
        a0 = jnp.pad(a[:16 * K0].reshape(16, K0, CH),
                     ((0, 0), (0, KMAX - K0), (0, 0)))
        a1 = a[16 * K0:].reshape(16, K1, CH)
        return jnp.concatenate([a0, a1], axis=0)

    src_p = split(src_f)
    dst_p = split(dst_f)

    degp = _sc_deg(dst_p)
    d0 = degp[:N].reshape(N, 1)
    d1 = degp[NPAD:NPAD + N].reshape(N, 1)

    xws, dinv = _dense0(x, W1, d0, d1)
    b1r = b1.reshape(1, D)
    b2r = b2.reshape(1, D)
    b3r = b3.reshape(1, D)
    batch_c = batch.reshape(N, 1)

    p = _sc_layer(src_p, dst_p, xws)
    xws = _dense_mid(p[:N], p[NPAD:NPAD + N], xws, dinv, b1r, W2)
    p = _sc_layer(src_p, dst_p, xws)
    xws = _dense_mid(p[:N], p[NPAD:NPAD + N], xws, dinv, b2r, W3)
    p = _sc_layer(src_p, dst_p, xws)
    out = _dense_last(p[:N], p[NPAD:NPAD + N], xws, dinv, b3r, batch_c)
    return out


# speedup vs baseline: 1.0680x; 1.0680x over previous
import functools

import jax
import jax.numpy as jnp
from jax import lax
from jax.experimental import pallas as pl
from jax.experimental.pallas import tpu as pltpu
from jax.experimental.pallas import tpu_sc as plsc

N = 10000
D = 128
G = 64
NPAD = 10112
RPT = NPAD // 16
CH = 128
TOT = 2560
EPAD = TOT * CH
K0 = 112
K1 = 48
KMAX = 112
IDXR = 16
NSTG = KMAX // IDXR
BLK = 1000

_mesh = lambda: plsc.VectorSubcoreMesh(core_axis_name="c", subcore_axis_name="s")


@functools.partial(
    pl.kernel,
    mesh=_mesh(),
    out_type=jax.ShapeDtypeStruct((2 * NPAD,), jnp.float32),
    scratch_types=[
        pltpu.VMEM_SHARED((NPAD,), jnp.float32),
        pltpu.VMEM((KMAX, CH), jnp.int32),
        pltpu.VMEM((CH,), jnp.float32),
        pltpu.VMEM((640,), jnp.float32),
    ],
)
def _sc_deg(dst_hbm, out_hbm, acc, dst_v, ones_v, zb_v):
    c = lax.axis_index("c")
    s = lax.axis_index("s")
    wid = c * 16 + s
    kc = jnp.where(c == 0, K0, K1)

    def fill(i, _):
        ones_v[pl.ds(i * 16, 16)] = jnp.ones((16,), jnp.float32)
        zb_v[pl.ds(i * 16, 16)] = jnp.zeros((16,), jnp.float32)
        return 0

    lax.fori_loop(0, 8, fill, 0)

    def fillz(i, _):
        zb_v[pl.ds(i * 16, 16)] = jnp.zeros((16,), jnp.float32)
        return 0

    lax.fori_loop(8, 40, fillz, 0)

    pltpu.sync_copy(dst_hbm.at[wid], dst_v)
    base = s * RPT
    pltpu.sync_copy(zb_v.at[pl.ds(0, RPT)], acc.at[pl.ds(base, RPT)])
    plsc.subcore_barrier()

    def body(j, _):
        pltpu.sync_copy(ones_v, acc.at[dst_v.at[j]], add=True)
        return 0

    lax.fori_loop(0, kc, body, 0)
    plsc.subcore_barrier()

    pltpu.sync_copy(acc.at[pl.ds(base, RPT)], zb_v.at[pl.ds(0, RPT)])
    pltpu.sync_copy(zb_v.at[pl.ds(0, RPT)], out_hbm.at[pl.ds(c * NPAD + base, RPT)])


@functools.partial(
    pl.kernel,
    mesh=_mesh(),
    out_type=jax.ShapeDtypeStruct((2 * NPAD, D), jnp.float32),
    scratch_types=[
        pltpu.VMEM_SHARED((NPAD, D), jnp.float32),
        pltpu.VMEM((IDXR, CH), jnp.int32),
        pltpu.VMEM((IDXR, CH), jnp.int32),
        pltpu.VMEM((CH, D), jnp.float32),
        pltpu.VMEM((CH, D), jnp.float32),
        pltpu.SemaphoreType.DMA,
        pltpu.SemaphoreType.DMA,
    ],
)
def _sc_layer(src_hbm, dst_hbm, table_hbm, out_hbm, acc, src_v, dst_v,
              rows0_v, rows1_v, sem0, sem1):
    c = lax.axis_index("c")
    s = lax.axis_index("s")
    wid = c * 16 + s
    base = s * RPT

    def zrow(i, _):
        r = i // 8
        g = i % 8
        rows0_v[r, pl.ds(g * 16, 16)] = jnp.zeros((16,), jnp.float32)
        return 0

    lax.fori_loop(0, CH * 8, zrow, 0)
    for k in range(4):
        pltpu.sync_copy(rows0_v, acc.at[pl.ds(base + k * CH, CH)])
    tail = RPT - 4 * CH
    pltpu.sync_copy(rows0_v.at[pl.ds(0, tail)],
                    acc.at[pl.ds(base + 4 * CH, tail)])
    plsc.subcore_barrier()

    kc = jnp.where(c == 0, K0, K1)
    for st in range(NSTG):
        @pl.when(st * IDXR < kc)
        def _stage():
            pltpu.sync_copy(src_hbm.at[wid, pl.ds(st * IDXR, IDXR)], src_v)
            pltpu.sync_copy(dst_hbm.at[wid, pl.ds(st * IDXR, IDXR)], dst_v)
            pltpu.async_copy(table_hbm.at[src_v.at[0]], rows0_v, sem0)
            pltpu.async_copy(table_hbm.at[src_v.at[1]], rows1_v, sem1)

            def body(i, _):
                j0 = 2 * i
                j1 = j0 + 1
                pltpu.make_async_copy(
                    table_hbm.at[src_v.at[j0]], rows0_v, sem0).wait()
                pltpu.sync_copy(rows0_v, acc.at[dst_v.at[j0]], add=True)

                @pl.when(j0 + 2 < IDXR)
                def _():
                    pltpu.async_copy(
                        table_hbm.at[src_v.at[j0 + 2]], rows0_v, sem0)

                pltpu.make_async_copy(
                    table_hbm.at[src_v.at[j1]], rows1_v, sem1).wait()
                pltpu.sync_copy(rows1_v, acc.at[dst_v.at[j1]], add=True)

                @pl.when(j1 + 2 < IDXR)
                def _():
                    pltpu.async_copy(
                        table_hbm.at[src_v.at[j1 + 2]], rows1_v, sem1)

                return 0

            lax.fori_loop(0, IDXR // 2, body, 0)
    plsc.subcore_barrier()

    for k in range(4):
        pltpu.sync_copy(acc.at[pl.ds(base + k * CH, CH)], rows0_v)
        pltpu.sync_copy(rows0_v, out_hbm.at[pl.ds(c * NPAD + base + k * CH, CH)])
    pltpu.sync_copy(acc.at[pl.ds(base + 4 * CH, tail)],
                    rows0_v.at[pl.ds(0, tail)])
    pltpu.sync_copy(rows0_v.at[pl.ds(0, tail)],
                    out_hbm.at[pl.ds(c * NPAD + base + 4 * CH, tail)])


def _dense0_body(x_ref, w_ref, d0_ref, d1_ref, out_ref, dinv_ref):
    dinv = lax.rsqrt(1.0 + d0_ref[...] + d1_ref[...])
    out_ref[...] = dinv * jnp.dot(
        x_ref[...], w_ref[...], preferred_element_type=jnp.float32)
    dinv_ref[...] = dinv


def _dense_mid_body(p0_ref, p1_ref, xws_ref, dinv_ref, b_ref, w_ref, out_ref):
    dinv = dinv_ref[...]
    h = jnp.maximum(
        dinv * (p0_ref[...] + p1_ref[...] + xws_ref[...]) + b_ref[...], 0.0)
    out_ref[...] = dinv * jnp.dot(
        h, w_ref[...], preferred_element_type=jnp.float32)


def _dense_last_body(p0_ref, p1_ref, xws_ref, dinv_ref, b_ref, batch_ref, out_ref):
    dinv = dinv_ref[...]
    h = jnp.maximum(
        dinv * (p0_ref[...] + p1_ref[...] + xws_ref[...]) + b_ref[...], 0.0)
    gids = lax.broadcasted_iota(jnp.int32, (BLK, G), 1)
    onehot = (batch_ref[...] == gids).astype(jnp.float32)
    contrib = lax.dot_general(
        onehot, h, (((0,), (0,)), ((), ())),
        preferred_element_type=jnp.float32)

    @pl.when(pl.program_id(0) == 0)
    def _():
        out_ref[...] = jnp.zeros_like(out_ref)

    out_ref[...] += contrib


_row_spec = pl.BlockSpec((BLK, D), lambda i: (i, 0))
_col_spec = pl.BlockSpec((BLK, 1), lambda i: (i, 0))
_w_spec = pl.BlockSpec((D, D), lambda i: (0, 0))
_b_spec = pl.BlockSpec((1, D), lambda i: (0, 0))
_p_spec = pl.BlockSpec((2, BLK, D), lambda i: (0, i, 0))

_GRID = N // BLK

_dense0 = pl.pallas_call(
    _dense0_body,
    grid=(_GRID,),
    in_specs=[_row_spec, _w_spec, _col_spec, _col_spec],
    out_specs=[_row_spec, _col_spec],
    out_shape=[
        jax.ShapeDtypeStruct((N, D), jnp.float32),
        jax.ShapeDtypeStruct((N, 1), jnp.float32),
    ],
)

_dense_mid = pl.pallas_call(
    _dense_mid_body,
    grid=(_GRID,),
    in_specs=[_row_spec, _row_spec, _row_spec, _col_spec, _b_spec, _w_spec],
    out_specs=_row_spec,
    out_shape=jax.ShapeDtypeStruct((N, D), jnp.float32),
)

_dense_last = pl.pallas_call(
    _dense_last_body,
    grid=(_GRID,),
    in_specs=[_row_spec, _row_spec, _row_spec, _col_spec, _b_spec,
              pl.BlockSpec((BLK, 1), lambda i: (i, 0))],
    out_specs=pl.BlockSpec((G, D), lambda i: (0, 0)),
    out_shape=jax.ShapeDtypeStruct((G, D), jnp.float32),
)


@jax.jit
def kernel(x, edge_index, batch, W1, b1, W2, b2, W3, b3):
    src = edge_index[0]
    dst = edge_index[1]
    pad = EPAD - src.shape[0]
    src_f = jnp.concatenate(
        [src, jnp.zeros((pad,), jnp.int32)]).reshape(TOT, CH)
    dummy = N + (jnp.arange(pad, dtype=jnp.int32) % (NPAD - N))
    dst_f = jnp.concatenate([dst, dummy]).reshape(TOT, CH)

    def split(a):
        a0 = jnp.pad(a[:16 * K0].reshape(16, K0, CH),
                     ((0, 0), (0, KMAX - K0), (0, 0)))
        a1 = jnp.pad(a[16 * K0:].reshape(16, K1, CH),
                     ((0, 0), (0, KMAX - K1), (0, 0)))
        return jnp.concatenate([a0, a1], axis=0)

    src_p = split(src_f)
    dst_p = split(dst_f)

    degp = _sc_deg(dst_p)
    d0 = degp[:N].reshape(N, 1)
    d1 = degp[NPAD:NPAD + N].reshape(N, 1)

    xws, dinv = _dense0(x, W1, d0, d1)
    b1r = b1.reshape(1, D)
    b2r = b2.reshape(1, D)
    b3r = b3.reshape(1, D)
    batch_c = batch.reshape(N, 1)

    p = _sc_layer(src_p, dst_p, xws)
    xws = _dense_mid(p[:N], p[NPAD:NPAD + N], xws, dinv, b1r, W2)
    p = _sc_layer(src_p, dst_p, xws)
    xws = _dense_mid(p[:N], p[NPAD:NPAD + N], xws, dinv, b2r, W3)
    p = _sc_layer(src_p, dst_p, xws)
    out = _dense_last(p[:N], p[NPAD:NPAD + N], xws, dinv, b3r, batch_c)
    return out


# speedup vs baseline: 1.1986x; 1.1223x over previous
import functools

import jax
import jax.numpy as jnp
from jax import lax
from jax.experimental import pallas as pl
from jax.experimental.pallas import tpu as pltpu
from jax.experimental.pallas import tpu_sc as plsc

N = 10000
D = 128
G = 64
NPAD = 10112
RPT = NPAD // 16
CH = 128
K = 80
EPT = K * CH
EPAD = 32 * EPT
IDXR = 16
NSTG = K // IDXR
BLK = 1000

_mesh = lambda: plsc.VectorSubcoreMesh(core_axis_name="c", subcore_axis_name="s")


@functools.partial(
    pl.kernel,
    mesh=_mesh(),
    out_type=jax.ShapeDtypeStruct((2 * NPAD,), jnp.float32),
    scratch_types=[
        pltpu.VMEM_SHARED((NPAD,), jnp.float32),
        pltpu.VMEM((K, CH), jnp.int32),
        pltpu.VMEM((CH,), jnp.float32),
        pltpu.VMEM((640,), jnp.float32),
    ],
)
def _sc_deg(dst_hbm, out_hbm, acc, dst_v, ones_v, zb_v):
    c = lax.axis_index("c")
    s = lax.axis_index("s")
    wid = c * 16 + s

    def fill(i, _):
        ones_v[pl.ds(i * 16, 16)] = jnp.ones((16,), jnp.float32)
        zb_v[pl.ds(i * 16, 16)] = jnp.zeros((16,), jnp.float32)
        return 0

    lax.fori_loop(0, 8, fill, 0)

    def fillz(i, _):
        zb_v[pl.ds(i * 16, 16)] = jnp.zeros((16,), jnp.float32)
        return 0

    lax.fori_loop(8, 40, fillz, 0)

    pltpu.sync_copy(dst_hbm.at[wid], dst_v)
    base = s * RPT
    pltpu.sync_copy(zb_v.at[pl.ds(0, RPT)], acc.at[pl.ds(base, RPT)])
    plsc.subcore_barrier()

    def body(j, _):
        pltpu.sync_copy(ones_v, acc.at[dst_v.at[j]], add=True)
        return 0

    lax.fori_loop(0, K, body, 0)
    plsc.subcore_barrier()

    pltpu.sync_copy(acc.at[pl.ds(base, RPT)], zb_v.at[pl.ds(0, RPT)])
    pltpu.sync_copy(zb_v.at[pl.ds(0, RPT)], out_hbm.at[pl.ds(c * NPAD + base, RPT)])


@functools.partial(
    pl.kernel,
    mesh=_mesh(),
    out_type=jax.ShapeDtypeStruct((2 * NPAD, D), jnp.float32),
    scratch_types=[
        pltpu.VMEM_SHARED((NPAD, D), jnp.float32),
        pltpu.VMEM((IDXR, CH), jnp.int32),
        pltpu.VMEM((IDXR, CH), jnp.int32),
        pltpu.VMEM((CH, D), jnp.float32),
        pltpu.VMEM((CH, D), jnp.float32),
        pltpu.SemaphoreType.DMA,
        pltpu.SemaphoreType.DMA,
    ],
)
def _sc_layer(src_hbm, dst_hbm, table_hbm, out_hbm, acc, src_v, dst_v,
              rows0_v, rows1_v, sem0, sem1):
    c = lax.axis_index("c")
    s = lax.axis_index("s")
    wid = c * 16 + s
    base = s * RPT

    def zrow(i, _):
        r = i // 8
        g = i % 8
        rows0_v[r, pl.ds(g * 16, 16)] = jnp.zeros((16,), jnp.float32)
        return 0

    lax.fori_loop(0, CH * 8, zrow, 0)
    for k in range(4):
        pltpu.sync_copy(rows0_v, acc.at[pl.ds(base + k * CH, CH)])
    tail = RPT - 4 * CH
    pltpu.sync_copy(rows0_v.at[pl.ds(0, tail)],
                    acc.at[pl.ds(base + 4 * CH, tail)])
    plsc.subcore_barrier()

    for st in range(NSTG):
        pltpu.sync_copy(src_hbm.at[wid, pl.ds(st * IDXR, IDXR)], src_v)
        pltpu.sync_copy(dst_hbm.at[wid, pl.ds(st * IDXR, IDXR)], dst_v)
        pltpu.async_copy(table_hbm.at[src_v.at[0]], rows0_v, sem0)
        pltpu.async_copy(table_hbm.at[src_v.at[1]], rows1_v, sem1)

        def body(i, _):
            j0 = 2 * i
            j1 = j0 + 1
            pltpu.make_async_copy(
                table_hbm.at[src_v.at[j0]], rows0_v, sem0).wait()
            pltpu.sync_copy(rows0_v, acc.at[dst_v.at[j0]], add=True)

            @pl.when(j0 + 2 < IDXR)
            def _():
                pltpu.async_copy(table_hbm.at[src_v.at[j0 + 2]], rows0_v, sem0)

            pltpu.make_async_copy(
                table_hbm.at[src_v.at[j1]], rows1_v, sem1).wait()
            pltpu.sync_copy(rows1_v, acc.at[dst_v.at[j1]], add=True)

            @pl.when(j1 + 2 < IDXR)
            def _():
                pltpu.async_copy(table_hbm.at[src_v.at[j1 + 2]], rows1_v, sem1)

            return 0

        lax.fori_loop(0, IDXR // 2, body, 0)
    plsc.subcore_barrier()

    for k in range(4):
        pltpu.sync_copy(acc.at[pl.ds(base + k * CH, CH)], rows0_v)
        pltpu.sync_copy(rows0_v, out_hbm.at[pl.ds(c * NPAD + base + k * CH, CH)])
    pltpu.sync_copy(acc.at[pl.ds(base + 4 * CH, tail)],
                    rows0_v.at[pl.ds(0, tail)])
    pltpu.sync_copy(rows0_v.at[pl.ds(0, tail)],
                    out_hbm.at[pl.ds(c * NPAD + base + 4 * CH, tail)])


def _dense0_body(x_ref, w_ref, d0_ref, d1_ref, out_ref, dinv_ref):
    dinv = lax.rsqrt(1.0 + d0_ref[...] + d1_ref[...])
    out_ref[...] = dinv * jnp.dot(
        x_ref[...], w_ref[...], preferred_element_type=jnp.float32)
    dinv_ref[...] = dinv


def _dense_mid_body(p0_ref, p1_ref, xws_ref, dinv_ref, b_ref, w_ref, out_ref):
    dinv = dinv_ref[...]
    h = jnp.maximum(
        dinv * (p0_ref[...] + p1_ref[...] + xws_ref[...]) + b_ref[...], 0.0)
    out_ref[...] = dinv * jnp.dot(
        h, w_ref[...], preferred_element_type=jnp.float32)


def _dense_last_body(p0_ref, p1_ref, xws_ref, dinv_ref, b_ref, batch_ref, out_ref):
    dinv = dinv_ref[...]
    h = jnp.maximum(
        dinv * (p0_ref[...] + p1_ref[...] + xws_ref[...]) + b_ref[...], 0.0)
    gids = lax.broadcasted_iota(jnp.int32, (BLK, G), 1)
    onehot = (batch_ref[...] == gids).astype(jnp.float32)
    contrib = lax.dot_general(
        onehot, h, (((0,), (0,)), ((), ())),
        preferred_element_type=jnp.float32)

    @pl.when(pl.program_id(0) == 0)
    def _():
        out_ref[...] = jnp.zeros_like(out_ref)

    out_ref[...] += contrib


_row_spec = pl.BlockSpec((BLK, D), lambda i: (i, 0))
_col_spec = pl.BlockSpec((BLK, 1), lambda i: (i, 0))
_w_spec = pl.BlockSpec((D, D), lambda i: (0, 0))
_b_spec = pl.BlockSpec((1, D), lambda i: (0, 0))
_p_spec = pl.BlockSpec((2, BLK, D), lambda i: (0, i, 0))

_GRID = N // BLK

_dense0 = pl.pallas_call(
    _dense0_body,
    grid=(_GRID,),
    in_specs=[_row_spec, _w_spec, _col_spec, _col_spec],
    out_specs=[_row_spec, _col_spec],
    out_shape=[
        jax.ShapeDtypeStruct((N, D), jnp.float32),
        jax.ShapeDtypeStruct((N, 1), jnp.float32),
    ],
)

_dense_mid = pl.pallas_call(
    _dense_mid_body,
    grid=(_GRID,),
    in_specs=[_row_spec, _row_spec, _row_spec, _col_spec, _b_spec, _w_spec],
    out_specs=_row_spec,
    out_shape=jax.ShapeDtypeStruct((N, D), jnp.float32),
)

_dense_last = pl.pallas_call(
    _dense_last_body,
    grid=(_GRID,),
    in_specs=[_row_spec, _row_spec, _row_spec, _col_spec, _b_spec,
              pl.BlockSpec((BLK, 1), lambda i: (i, 0))],
    out_specs=pl.BlockSpec((G, D), lambda i: (0, 0)),
    out_shape=jax.ShapeDtypeStruct((G, D), jnp.float32),
)


@jax.jit
def kernel(x, edge_index, batch, W1, b1, W2, b2, W3, b3):
    src = edge_index[0]
    dst = edge_index[1]
    pad = EPAD - src.shape[0]
    src_p = jnp.concatenate(
        [src, jnp.zeros((pad,), jnp.int32)]).reshape(32, K, CH)
    dummy = N + (jnp.arange(pad, dtype=jnp.int32) % (NPAD - N))
    dst_p = jnp.concatenate([dst, dummy]).reshape(32, K, CH)

    degp = _sc_deg(dst_p)
    d0 = degp[:N].reshape(N, 1)
    d1 = degp[NPAD:NPAD + N].reshape(N, 1)

    xws, dinv = _dense0(x, W1, d0, d1)
    b1r = b1.reshape(1, D)
    b2r = b2.reshape(1, D)
    b3r = b3.reshape(1, D)
    batch_c = batch.reshape(N, 1)

    p = _sc_layer(src_p, dst_p, xws)
    xws = _dense_mid(p[:N], p[NPAD:NPAD + N], xws, dinv, b1r, W2)
    p = _sc_layer(src_p, dst_p, xws)
    xws = _dense_mid(p[:N], p[NPAD:NPAD + N], xws, dinv, b2r, W3)
    p = _sc_layer(src_p, dst_p, xws)
    out = _dense_last(p[:N], p[NPAD:NPAD + N], xws, dinv, b3r, batch_c)
    return out
